# baseline (speedup 1.0000x reference)
import functools

import jax
import jax.numpy as jnp
from jax.experimental import pallas as pl
from jax.experimental.pallas import tpu as pltpu
from jax.experimental.pallas import tpu_sc as plsc

_N = 4096
_DIM = 128
_K = 20
_ALPHA = 3.0
_BLK = 512


def _gather_body(table_hbm, idx_hbm, out_hbm, idx_v, rows_v, sem, *, n_cores, b_per_w):
    wid = jax.lax.axis_index("s") * n_cores + jax.lax.axis_index("c")
    base = wid * b_per_w
    pltpu.sync_copy(idx_hbm.at[pl.ds(base, b_per_w)], idx_v)
    pltpu.async_copy(table_hbm.at[idx_v], rows_v, sem).wait()
    pltpu.sync_copy(rows_v, out_hbm.at[pl.ds(base, b_per_w)])


def _sc_gather(emb_table, idx):
    info = plsc.get_sparse_core_info()
    nc, ns = info.num_cores, info.num_subcores
    nw = nc * ns
    b = idx.shape[0]
    b_per_w = b // nw
    mesh = plsc.VectorSubcoreMesh(core_axis_name="c", subcore_axis_name="s")
    k = pl.kernel(
        functools.partial(_gather_body, n_cores=nc, b_per_w=b_per_w),
        mesh=mesh,
        out_type=jax.ShapeDtypeStruct((b, emb_table.shape[1]), jnp.float32),
        scratch_types=[
            pltpu.VMEM((b_per_w,), jnp.int32),
            pltpu.VMEM((b_per_w, emb_table.shape[1]), jnp.float32),
            pltpu.SemaphoreType.DMA,
        ],
    )
    return k(emb_table, idx)


def _tc_body(vec1r_ref, gemb_ref, w1_ref, b1_ref, out_ref, vec1_ref):
    i = pl.program_id(0)

    @pl.when(i == 0)
    def _():
        h = jax.lax.dot_general(
            vec1r_ref[...], w1_ref[...], (((1,), (1,)), ((), ())),
            preferred_element_type=jnp.float32)
        vec1_ref[...] = jnp.tanh(_ALPHA * (h + b1_ref[...]))

    vblk = vec1_ref[pl.ds(i * _BLK, _BLK), :]
    gblk = gemb_ref[pl.ds(i * _BLK, _BLK), :]
    p = jax.lax.dot_general(
        vblk, gemb_ref[...], (((1,), (1,)), ((), ())),
        preferred_element_type=jnp.float32)
    q = jax.lax.dot_general(
        gblk, vec1_ref[...], (((1,), (1,)), ((), ())),
        preferred_element_type=jnp.float32)
    adj = jnp.maximum(jnp.tanh(_ALPHA * (p - q)), 0.0)

    def _cond(carry):
        _, cnt, _ = carry
        return jnp.any(cnt < float(_K))

    def _body(carry):
        t, cnt, tie_cnt = carry
        active = cnt < float(_K)
        masked = jnp.where(adj < t, adj, -1.0)
        m = jnp.max(masked, axis=1, keepdims=True)
        c = jnp.sum(jnp.where(adj == m, 1.0, 0.0), axis=1, keepdims=True)
        t = jnp.where(active, m, t)
        cnt = jnp.where(active, cnt + c, cnt)
        tie_cnt = jnp.where(active, c, tie_cnt)
        return t, cnt, tie_cnt

    t0 = jnp.max(adj, axis=1, keepdims=True)
    c0 = jnp.sum(jnp.where(adj == t0, 1.0, 0.0), axis=1, keepdims=True)
    t, cnt, tie_cnt = t0, c0, c0

    ties = adj == t
    tiesf = jnp.where(ties, 1.0, 0.0)
    m20 = float(_K) - (cnt - tie_cnt)

    cur = tiesf
    base = jnp.zeros((_BLK, 1), jnp.float32)
    need = m20
    w = _N
    while w > 1:
        half = w // 2
        left = cur[:, :half]
        right = cur[:, half:w]
        cl = jnp.sum(left, axis=1, keepdims=True)
        goleft = need <= cl
        cur = jnp.where(goleft, left, right)
        base = jnp.where(goleft, base, base + float(half))
        need = jnp.where(goleft, need, need - cl)
        w = half
    c_last = base

    colsf = jax.lax.broadcasted_iota(jnp.int32, (_BLK, _N), 1).astype(jnp.float32)
    keep = (adj > t) | (ties & (colsf <= c_last))
    out_ref[...] = jnp.where(keep, adj, 0.0)


def _tc_graph(vec1_raw, gEmb, W1, b1):
    grid = _N // _BLK
    return pl.pallas_call(
        _tc_body,
        grid=(grid,),
        in_specs=[
            pl.BlockSpec((_N, _DIM), lambda i: (0, 0)),
            pl.BlockSpec((_N, _DIM), lambda i: (0, 0)),
            pl.BlockSpec((_DIM, _DIM), lambda i: (0, 0)),
            pl.BlockSpec((1, _DIM), lambda i: (0, 0)),
        ],
        out_specs=pl.BlockSpec((_BLK, _N), lambda i: (i, 0)),
        out_shape=jax.ShapeDtypeStruct((_N, _N), jnp.float32),
        scratch_shapes=[pltpu.VMEM((_N, _DIM), jnp.float32)],
    )(vec1_raw, gEmb, W1, b1)


def kernel(idx, gEmb, emb_table, W1, b1):
    idx = idx.astype(jnp.int32)
    vec1_raw = _sc_gather(emb_table, idx)
    return _tc_graph(vec1_raw, gEmb, W1, b1.reshape(1, _DIM))


# speedup vs baseline: 1.0594x; 1.0594x over previous
import functools

import jax
import jax.numpy as jnp
from jax.experimental import pallas as pl
from jax.experimental.pallas import tpu as pltpu
from jax.experimental.pallas import tpu_sc as plsc

_N = 4096
_DIM = 128
_K = 20
_ALPHA = 3.0
_BLK = 512


def _gather_body(table_hbm, idx_hbm, out_hbm, idx_v, rows_v, sem, *, n_cores, b_per_w):
    wid = jax.lax.axis_index("s") * n_cores + jax.lax.axis_index("c")
    base = wid * b_per_w
    pltpu.sync_copy(idx_hbm.at[pl.ds(base, b_per_w)], idx_v)
    pltpu.async_copy(table_hbm.at[idx_v], rows_v, sem).wait()
    pltpu.sync_copy(rows_v, out_hbm.at[pl.ds(base, b_per_w)])


def _sc_gather(emb_table, idx):
    info = plsc.get_sparse_core_info()
    nc, ns = info.num_cores, info.num_subcores
    nw = nc * ns
    b = idx.shape[0]
    b_per_w = b // nw
    mesh = plsc.VectorSubcoreMesh(core_axis_name="c", subcore_axis_name="s")
    k = pl.kernel(
        functools.partial(_gather_body, n_cores=nc, b_per_w=b_per_w),
        mesh=mesh,
        out_type=jax.ShapeDtypeStruct((b, emb_table.shape[1]), jnp.float32),
        scratch_types=[
            pltpu.VMEM((b_per_w,), jnp.int32),
            pltpu.VMEM((b_per_w, emb_table.shape[1]), jnp.float32),
            pltpu.SemaphoreType.DMA,
        ],
    )
    return k(emb_table, idx)


def _tc_body(vec1r_ref, gemb_ref, w1_ref, b1_ref, out_ref, vec1_ref):
    i = pl.program_id(0)

    @pl.when(i == 0)
    def _():
        h = jax.lax.dot_general(
            vec1r_ref[...], w1_ref[...], (((1,), (1,)), ((), ())),
            preferred_element_type=jnp.float32)
        vec1_ref[...] = jnp.tanh(_ALPHA * (h + b1_ref[...]))

    vblk = vec1_ref[pl.ds(i * _BLK, _BLK), :]
    gblk = gemb_ref[pl.ds(i * _BLK, _BLK), :]
    p = jax.lax.dot_general(
        vblk, gemb_ref[...], (((1,), (1,)), ((), ())),
        preferred_element_type=jnp.float32)
    q = jax.lax.dot_general(
        gblk, vec1_ref[...], (((1,), (1,)), ((), ())),
        preferred_element_type=jnp.float32)
    adj = jnp.maximum(jnp.tanh(_ALPHA * (p - q)), 0.0)

    def _cond(carry):
        _, cnt, _ = carry
        return jnp.any(cnt < float(_K))

    def _body(carry):
        t, cnt, tie_cnt = carry
        active = cnt < float(_K)
        masked = jnp.where(adj < t, adj, -1.0)
        m = jnp.max(masked, axis=1, keepdims=True)
        c = jnp.sum(jnp.where(adj == m, 1.0, 0.0), axis=1, keepdims=True)
        t = jnp.where(active, m, t)
        cnt = jnp.where(active, cnt + c, cnt)
        tie_cnt = jnp.where(active, c, tie_cnt)
        return t, cnt, tie_cnt

    t0 = jnp.max(adj, axis=1, keepdims=True)
    c0 = jnp.sum(jnp.where(adj == t0, 1.0, 0.0), axis=1, keepdims=True)
    t, cnt, tie_cnt = jax.lax.while_loop(_cond, _body, (t0, c0, c0))

    ties = adj == t
    tiesf = jnp.where(ties, 1.0, 0.0)
    m20 = float(_K) - (cnt - tie_cnt)

    cur = tiesf
    base = jnp.zeros((_BLK, 1), jnp.float32)
    need = m20
    w = _N
    while w > 1:
        half = w // 2
        left = cur[:, :half]
        right = cur[:, half:w]
        cl = jnp.sum(left, axis=1, keepdims=True)
        goleft = need <= cl
        cur = jnp.where(goleft, left, right)
        base = jnp.where(goleft, base, base + float(half))
        need = jnp.where(goleft, need, need - cl)
        w = half
    c_last = base

    cols_i = jax.lax.broadcasted_iota(jnp.int32, (_BLK, _N), 1)
    keep = (adj > t) | (ties & (cols_i <= c_last.astype(jnp.int32)))
    out_ref[...] = jnp.where(keep, adj, 0.0)


def _tc_graph(vec1_raw, gEmb, W1, b1):
    grid = _N // _BLK
    return pl.pallas_call(
        _tc_body,
        grid=(grid,),
        in_specs=[
            pl.BlockSpec((_N, _DIM), lambda i: (0, 0)),
            pl.BlockSpec((_N, _DIM), lambda i: (0, 0)),
            pl.BlockSpec((_DIM, _DIM), lambda i: (0, 0)),
            pl.BlockSpec((1, _DIM), lambda i: (0, 0)),
        ],
        out_specs=pl.BlockSpec((_BLK, _N), lambda i: (i, 0)),
        out_shape=jax.ShapeDtypeStruct((_N, _N), jnp.float32),
        scratch_shapes=[pltpu.VMEM((_N, _DIM), jnp.float32)],
    )(vec1_raw, gEmb, W1, b1)


def kernel(idx, gEmb, emb_table, W1, b1):
    idx = idx.astype(jnp.int32)
    vec1_raw = _sc_gather(emb_table, idx)
    return _tc_graph(vec1_raw, gEmb, W1, b1.reshape(1, _DIM))


# speedup vs baseline: 1.2041x; 1.1366x over previous
import functools

import jax
import jax.numpy as jnp
from jax.experimental import pallas as pl
from jax.experimental.pallas import tpu as pltpu
from jax.experimental.pallas import tpu_sc as plsc

_N = 4096
_DIM = 128
_K = 20
_ALPHA = 3.0
_BLK = 512


def _gather_body(table_hbm, idx_hbm, out_hbm, idx_v, rows_v, sem, *, n_cores, b_per_w):
    wid = jax.lax.axis_index("s") * n_cores + jax.lax.axis_index("c")
    base = wid * b_per_w
    pltpu.sync_copy(idx_hbm.at[pl.ds(base, b_per_w)], idx_v)
    pltpu.async_copy(table_hbm.at[idx_v], rows_v, sem).wait()
    pltpu.sync_copy(rows_v, out_hbm.at[pl.ds(base, b_per_w)])


def _sc_gather(emb_table, idx):
    info = plsc.get_sparse_core_info()
    nc, ns = info.num_cores, info.num_subcores
    nw = nc * ns
    b = idx.shape[0]
    b_per_w = b // nw
    mesh = plsc.VectorSubcoreMesh(core_axis_name="c", subcore_axis_name="s")
    k = pl.kernel(
        functools.partial(_gather_body, n_cores=nc, b_per_w=b_per_w),
        mesh=mesh,
        out_type=jax.ShapeDtypeStruct((b, emb_table.shape[1]), jnp.float32),
        scratch_types=[
            pltpu.VMEM((b_per_w,), jnp.int32),
            pltpu.VMEM((b_per_w, emb_table.shape[1]), jnp.float32),
            pltpu.SemaphoreType.DMA,
        ],
    )
    return k(emb_table, idx)


def _tc_body(vec1r_ref, gemb_ref, w1_ref, b1_ref, out_ref, u_ref, v_ref):
    i = pl.program_id(0)

    @pl.when(i == 0)
    def _():
        h = jax.lax.dot_general(
            vec1r_ref[...], w1_ref[...], (((1,), (1,)), ((), ())),
            preferred_element_type=jnp.float32)
        v1 = jnp.tanh(_ALPHA * (h + b1_ref[...]))
        g = gemb_ref[...]
        u_ref[:, :_DIM] = v1
        u_ref[:, _DIM:] = -g
        v_ref[:, :_DIM] = g
        v_ref[:, _DIM:] = v1

    ublk = u_ref[pl.ds(i * _BLK, _BLK), :]
    a = jax.lax.dot_general(
        ublk, v_ref[...], (((1,), (1,)), ((), ())),
        preferred_element_type=jnp.float32)
    adj = jnp.maximum(jnp.tanh(_ALPHA * a), 0.0)

    def _cond(carry):
        _, cnt, _ = carry
        return jnp.any(cnt < float(_K))

    def _body(carry):
        t, cnt, tie_cnt = carry
        active = cnt < float(_K)
        masked = jnp.where(adj < t, adj, -1.0)
        m = jnp.max(masked, axis=1, keepdims=True)
        c = jnp.sum(jnp.where(adj == m, 1.0, 0.0), axis=1, keepdims=True)
        t = jnp.where(active, m, t)
        cnt = jnp.where(active, cnt + c, cnt)
        tie_cnt = jnp.where(active, c, tie_cnt)
        return t, cnt, tie_cnt

    t0 = jnp.max(adj, axis=1, keepdims=True)
    c0 = jnp.sum(jnp.where(adj == t0, 1.0, 0.0), axis=1, keepdims=True)
    t, cnt, tie_cnt = jax.lax.while_loop(_cond, _body, (t0, c0, c0))

    ties = adj == t
    tiesf = jnp.where(ties, 1.0, 0.0)
    m20 = float(_K) - (cnt - tie_cnt)

    cur = tiesf
    base = jnp.zeros((_BLK, 1), jnp.float32)
    need = m20
    w = _N
    while w > 1:
        half = w // 2
        left = cur[:, :half]
        right = cur[:, half:w]
        cl = jnp.sum(left, axis=1, keepdims=True)
        goleft = need <= cl
        cur = jnp.where(goleft, left, right)
        base = jnp.where(goleft, base, base + float(half))
        need = jnp.where(goleft, need, need - cl)
        w = half
    c_last = base

    cols_i = jax.lax.broadcasted_iota(jnp.int32, (_BLK, _N), 1)
    keep = (adj > t) | (ties & (cols_i <= c_last.astype(jnp.int32)))
    out_ref[...] = jnp.where(keep, adj, 0.0)


def _tc_graph(vec1_raw, gEmb, W1, b1):
    grid = _N // _BLK
    return pl.pallas_call(
        _tc_body,
        grid=(grid,),
        in_specs=[
            pl.BlockSpec((_N, _DIM), lambda i: (0, 0)),
            pl.BlockSpec((_N, _DIM), lambda i: (0, 0)),
            pl.BlockSpec((_DIM, _DIM), lambda i: (0, 0)),
            pl.BlockSpec((1, _DIM), lambda i: (0, 0)),
        ],
        out_specs=pl.BlockSpec((_BLK, _N), lambda i: (i, 0)),
        out_shape=jax.ShapeDtypeStruct((_N, _N), jnp.float32),
        scratch_shapes=[pltpu.VMEM((_N, 2 * _DIM), jnp.float32),
                        pltpu.VMEM((_N, 2 * _DIM), jnp.float32)],
    )(vec1_raw, gEmb, W1, b1)


def kernel(idx, gEmb, emb_table, W1, b1):
    idx = idx.astype(jnp.int32)
    vec1_raw = _sc_gather(emb_table, idx)
    return _tc_graph(vec1_raw, gEmb, W1, b1.reshape(1, _DIM))


# speedup vs baseline: 1.4333x; 1.1904x over previous
import functools

import jax
import jax.numpy as jnp
from jax.experimental import pallas as pl
from jax.experimental.pallas import tpu as pltpu
from jax.experimental.pallas import tpu_sc as plsc

_N = 4096
_DIM = 128
_K = 20
_ALPHA = 3.0
_BLK = 512


def _gather_body(table_hbm, idx_hbm, out_hbm, idx_v, rows_v, sem, *, n_cores, b_per_w):
    wid = jax.lax.axis_index("s") * n_cores + jax.lax.axis_index("c")
    base = wid * b_per_w
    pltpu.sync_copy(idx_hbm.at[pl.ds(base, b_per_w)], idx_v)
    pltpu.async_copy(table_hbm.at[idx_v], rows_v, sem).wait()
    pltpu.sync_copy(rows_v, out_hbm.at[pl.ds(base, b_per_w)])


def _sc_gather(emb_table, idx):
    info = plsc.get_sparse_core_info()
    nc, ns = info.num_cores, info.num_subcores
    nw = nc * ns
    b = idx.shape[0]
    b_per_w = b // nw
    mesh = plsc.VectorSubcoreMesh(core_axis_name="c", subcore_axis_name="s")
    k = pl.kernel(
        functools.partial(_gather_body, n_cores=nc, b_per_w=b_per_w),
        mesh=mesh,
        out_type=jax.ShapeDtypeStruct((b, emb_table.shape[1]), jnp.float32),
        scratch_types=[
            pltpu.VMEM((b_per_w,), jnp.int32),
            pltpu.VMEM((b_per_w, emb_table.shape[1]), jnp.float32),
            pltpu.SemaphoreType.DMA,
        ],
    )
    return k(emb_table, idx)


def _tc_body(vec1r_ref, gemb_ref, w1_ref, b1_ref, out_ref, u_ref, v_ref):
    i = pl.program_id(0)

    @pl.when(i == 0)
    def _():
        h = jax.lax.dot_general(
            vec1r_ref[...], w1_ref[...], (((1,), (1,)), ((), ())),
            preferred_element_type=jnp.float32)
        v1 = jnp.tanh(_ALPHA * (h + b1_ref[...]))
        g = gemb_ref[...]
        u_ref[:, :_DIM] = v1
        u_ref[:, _DIM:] = -g
        v_ref[:, :_DIM] = g
        v_ref[:, _DIM:] = v1

    ublk = u_ref[pl.ds(i * _BLK, _BLK), :]
    a = jax.lax.dot_general(
        ublk, v_ref[...], (((1,), (1,)), ((), ())),
        preferred_element_type=jnp.float32)
    adj = jnp.maximum(jnp.tanh(_ALPHA * a), 0.0)

    t0 = jnp.max(adj, axis=1, keepdims=True)
    ties0 = adj == t0
    tiesf0 = jnp.where(ties0, 1.0, 0.0)
    cl1 = jnp.sum(tiesf0[:, :_N // 2], axis=1, keepdims=True)
    cr1 = jnp.sum(tiesf0[:, _N // 2:], axis=1, keepdims=True)
    c0 = cl1 + cr1

    cols_i = jax.lax.broadcasted_iota(jnp.int32, (_BLK, _N), 1)

    def _locate(tiesf, need, cl_first):
        goleft = need <= cl_first
        cur = jnp.where(goleft, tiesf[:, :_N // 2], tiesf[:, _N // 2:])
        base = jnp.where(goleft, 0.0, float(_N // 2))
        need = jnp.where(goleft, need, need - cl_first)
        w = _N // 2
        while w > 1:
            half = w // 2
            left = cur[:, :half]
            right = cur[:, half:w]
            cl = jnp.sum(left, axis=1, keepdims=True)
            goleft = need <= cl
            cur = jnp.where(goleft, left, right)
            base = jnp.where(goleft, base, base + float(half))
            need = jnp.where(goleft, need, need - cl)
            w = half
        return base.astype(jnp.int32)

    common = jnp.logical_not(jnp.any(c0 < float(_K)))

    @pl.when(common)
    def _():
        need0 = jnp.full((_BLK, 1), float(_K), jnp.float32)
        c_last = _locate(tiesf0, need0, cl1)
        keep = ties0 & (cols_i <= c_last)
        out_ref[...] = jnp.where(keep, t0, 0.0)

    @pl.when(jnp.logical_not(common))
    def _():
        def _cond(carry):
            _, cnt, _ = carry
            return jnp.any(cnt < float(_K))

        def _body(carry):
            t, cnt, tie_cnt = carry
            active = cnt < float(_K)
            masked = jnp.where(adj < t, adj, -1.0)
            m = jnp.max(masked, axis=1, keepdims=True)
            c = jnp.sum(jnp.where(adj == m, 1.0, 0.0), axis=1, keepdims=True)
            t = jnp.where(active, m, t)
            cnt = jnp.where(active, cnt + c, cnt)
            tie_cnt = jnp.where(active, c, tie_cnt)
            return t, cnt, tie_cnt

        t, cnt, tie_cnt = jax.lax.while_loop(_cond, _body, (t0, c0, c0))
        ties = adj == t
        tiesf = jnp.where(ties, 1.0, 0.0)
        m20 = float(_K) - (cnt - tie_cnt)
        clr = jnp.sum(tiesf[:, :_N // 2], axis=1, keepdims=True)
        c_last = _locate(tiesf, m20, clr)
        keep = (adj > t) | (ties & (cols_i <= c_last))
        out_ref[...] = jnp.where(keep, adj, 0.0)


def _tc_graph(vec1_raw, gEmb, W1, b1):
    grid = _N // _BLK
    return pl.pallas_call(
        _tc_body,
        grid=(grid,),
        in_specs=[
            pl.BlockSpec((_N, _DIM), lambda i: (0, 0)),
            pl.BlockSpec((_N, _DIM), lambda i: (0, 0)),
            pl.BlockSpec((_DIM, _DIM), lambda i: (0, 0)),
            pl.BlockSpec((1, _DIM), lambda i: (0, 0)),
        ],
        out_specs=pl.BlockSpec((_BLK, _N), lambda i: (i, 0)),
        out_shape=jax.ShapeDtypeStruct((_N, _N), jnp.float32),
        scratch_shapes=[pltpu.VMEM((_N, 2 * _DIM), jnp.float32),
                        pltpu.VMEM((_N, 2 * _DIM), jnp.float32)],
    )(vec1_raw, gEmb, W1, b1)


def kernel(idx, gEmb, emb_table, W1, b1):
    idx = idx.astype(jnp.int32)
    vec1_raw = _sc_gather(emb_table, idx)
    return _tc_graph(vec1_raw, gEmb, W1, b1.reshape(1, _DIM))
